# baseline (speedup 1.0000x reference)
import functools

import jax
import jax.numpy as jnp
from jax import lax
from jax.experimental import pallas as pl
from jax.experimental.pallas import tpu as pltpu
from jax.experimental.pallas import tpu_sc as plsc

CROP_H = 7
CROP_W = 7
NCELL = CROP_H * CROP_W
L = 16


def _sc_crop_and_resize(table, boxes_pad, B, C, H, W, N, npb, nw):
    nvr = C // L
    mesh = plsc.VectorSubcoreMesh(core_axis_name="c", subcore_axis_name="s")
    nc = mesh.num_cores

    @functools.partial(
        pl.kernel,
        out_type=jax.ShapeDtypeStruct((N, C * NCELL // 128, 128), jnp.float32),
        mesh=mesh,
        scratch_types=[
            pltpu.VMEM((npb, L), jnp.float32),
            pltpu.VMEM((8 * L,), jnp.int32),
            pltpu.VMEM((8 * L, C), jnp.float32),
            pltpu.VMEM((4, 4 * L), jnp.float32),
            pltpu.VMEM((C * NCELL // 128, 128), jnp.float32),
            pltpu.SemaphoreType.DMA,
        ],
        compiler_params=pltpu.CompilerParams(needs_layout_passes=False),
    )
    def k(table_hbm, boxes_hbm, out_hbm, boxes_v, idx_v, rows_v, wb_v,
          out_v, sem):
        wid = lax.axis_index("s") * nc + lax.axis_index("c")
        pltpu.sync_copy(boxes_hbm.at[pl.ds(wid * npb, npb)], boxes_v)
        lane = lax.iota(jnp.int32, L)
        zero = jnp.zeros((L,), jnp.int32)
        lane49 = lane * NCELL
        fH = jnp.float32(H - 1)
        fW = jnp.float32(W - 1)

        def box_body(b, carry):
            bs = jnp.broadcast_to(b, (L,))
            y1 = plsc.load_gather(boxes_v, [bs, zero])
            x1 = plsc.load_gather(boxes_v, [bs, zero + 1])
            y2 = plsc.load_gather(boxes_v, [bs, zero + 2])
            x2 = plsc.load_gather(boxes_v, [bs, zero + 3])
            bif = plsc.load_gather(boxes_v, [bs, zero + 4])
            bi = jnp.clip(bif.astype(jnp.int32), 0, B - 1)

            for half in range(2):
                for g2 in range(2):
                    g = half * 2 + g2
                    cell = jnp.broadcast_to(g * L, (L,)) + lane
                    iy = (cell * 147) >> 10
                    ix = cell - iy * CROP_W
                    ys = iy.astype(jnp.float32) / jnp.float32(CROP_H - 1)
                    xs = ix.astype(jnp.float32) / jnp.float32(CROP_W - 1)
                    in_y = (y1 + ys * (y2 - y1)) * fH
                    in_x = (x1 + xs * (x2 - x1)) * fW
                    valid = ((in_y >= 0.0) & (in_y <= fH)
                             & (in_x >= 0.0) & (in_x <= fW))

                    ty0 = in_y.astype(jnp.int32)
                    tf0 = ty0.astype(jnp.float32)
                    neg = in_y < tf0
                    fy_i = jnp.where(neg, ty0 - 1, ty0)
                    fy_f = jnp.where(neg, tf0 - 1.0, tf0)
                    ly = in_y - fy_f
                    ti = jnp.clip(fy_i, 0, H - 1)
                    bo = jnp.clip(fy_i + 1, 0, H - 1)

                    tx0 = in_x.astype(jnp.int32)
                    xf0 = tx0.astype(jnp.float32)
                    negx = in_x < xf0
                    fx_i = jnp.where(negx, tx0 - 1, tx0)
                    fx_f = jnp.where(negx, xf0 - 1.0, xf0)
                    lx = in_x - fx_f
                    li = jnp.clip(fx_i, 0, W - 1)
                    ri = jnp.clip(fx_i + 1, 0, W - 1)

                    vf = jnp.where(valid, 1.0, 0.0).astype(jnp.float32)
                    omy = 1.0 - ly
                    omx = 1.0 - lx
                    wb_v[0, pl.ds(g * L, L)] = omy * omx * vf
                    wb_v[1, pl.ds(g * L, L)] = omy * lx * vf
                    wb_v[2, pl.ds(g * L, L)] = ly * omx * vf
                    wb_v[3, pl.ds(g * L, L)] = ly * lx * vf

                    trow = bi * (H * W) + ti * W
                    brow = bi * (H * W) + bo * W
                    o = g2 * 4 * L
                    idx_v[pl.ds(o, L)] = trow + li
                    idx_v[pl.ds(o + L, L)] = trow + ri
                    idx_v[pl.ds(o + 2 * L, L)] = brow + li
                    idx_v[pl.ds(o + 3 * L, L)] = brow + ri

                pltpu.async_copy(table_hbm.at[idx_v], rows_v, sem).wait()

                for g2 in range(2):
                    g = half * 2 + g2
                    o = g2 * 4 * L

                    def cell_body(j, c, g=g, o=o):
                        col = jnp.broadcast_to(g * L, (L,)) + j
                        js = col
                        wtl = plsc.load_gather(wb_v, [zero, js])
                        wtr = plsc.load_gather(wb_v, [zero + 1, js])
                        wbl = plsc.load_gather(wb_v, [zero + 2, js])
                        wbr = plsc.load_gather(wb_v, [zero + 3, js])
                        msk = col < NCELL
                        off = lane49 + col
                        for kk in range(nvr):
                            sl = pl.ds(kk * L, L)
                            val = (wtl * rows_v[o + j, sl]
                                   + wtr * rows_v[o + j + L, sl]
                                   + wbl * rows_v[o + j + 2 * L, sl]
                                   + wbr * rows_v[o + j + 3 * L, sl])
                            plsc.store_scatter(
                                out_v, [off >> 7, off & 127], val,
                                mask=msk)
                            off = off + L * NCELL
                        return c

                    lax.fori_loop(0, L, cell_body, 0)

            n = wid * npb + b

            @pl.when(n < N)
            def _():
                pltpu.sync_copy(out_v, out_hbm.at[n])

            return carry

        lax.fori_loop(0, npb, box_body, 0)

    return k(table, boxes_pad)


def kernel(image, boxes, box_ind):
    B, C, H, W = image.shape
    N = boxes.shape[0]
    nw = 32
    npb = -(-N // (nw * 8)) * 8
    npad = npb * nw

    table = jnp.transpose(image, (0, 2, 3, 1)).reshape(B * H * W, C)
    boxes5 = jnp.concatenate(
        [boxes, box_ind[:, None].astype(jnp.float32)], axis=1)
    boxes_pad = jnp.zeros((npad, L), jnp.float32).at[:N, :5].set(boxes5)

    out = _sc_crop_and_resize(table, boxes_pad, B, C, H, W, N, npb, nw)
    return out.reshape(N, C, CROP_H, CROP_W)


# speedup vs baseline: 1.4977x; 1.4977x over previous
import functools

import jax
import jax.numpy as jnp
from jax import lax
from jax.experimental import pallas as pl
from jax.experimental.pallas import tpu as pltpu
from jax.experimental.pallas import tpu_sc as plsc

CROP_H = 7
CROP_W = 7
NCELL = CROP_H * CROP_W
L = 16


def _sc_crop_and_resize(table, boxes_pad, B, C, H, W, N, npb, nw):
    nvr = C // L
    mesh = plsc.VectorSubcoreMesh(core_axis_name="c", subcore_axis_name="s")
    nc = mesh.num_cores

    @functools.partial(
        pl.kernel,
        out_type=jax.ShapeDtypeStruct((N, C * NCELL // 128, 128),
                                      jnp.float32),
        mesh=mesh,
        scratch_types=[
            pltpu.VMEM((npb, L), jnp.float32),
            pltpu.VMEM((8 * L,), jnp.int32),
            pltpu.VMEM((5 * L,), jnp.int32),
            pltpu.VMEM((8 * L, C), jnp.float32),
            pltpu.VMEM((5 * L, C), jnp.float32),
            pltpu.VMEM((4, 4 * L), jnp.float32),
            pltpu.VMEM((C * NCELL // 128, 128), jnp.float32),
            pltpu.SemaphoreType.DMA,
            pltpu.SemaphoreType.DMA,
        ],
        compiler_params=pltpu.CompilerParams(needs_layout_passes=False),
    )
    def k(table_hbm, boxes_hbm, out_hbm, boxes_v, idxa, idxb, rowsa, rowsb,
          wb_v, out_v, sema, semb):
        wid = lax.axis_index("s") * nc + lax.axis_index("c")
        pltpu.sync_copy(boxes_hbm.at[pl.ds(wid * npb, npb)], boxes_v)
        lane = lax.iota(jnp.int32, L)
        zero = jnp.zeros((L,), jnp.int32)
        lane49 = lane * NCELL
        fH = jnp.float32(H - 1)
        fW = jnp.float32(W - 1)

        def load_box(b):
            bs = jnp.broadcast_to(b, (L,))
            y1 = plsc.load_gather(boxes_v, [bs, zero])
            x1 = plsc.load_gather(boxes_v, [bs, zero + 1])
            y2 = plsc.load_gather(boxes_v, [bs, zero + 2])
            x2 = plsc.load_gather(boxes_v, [bs, zero + 3])
            bif = plsc.load_gather(boxes_v, [bs, zero + 4])
            bi = jnp.clip(bif.astype(jnp.int32), 0, B - 1)
            return y1, x1, y2, x2, bi

        def cell_geom(cell, box):
            y1, x1, y2, x2, bi = box
            iy = (cell * 147) >> 10
            ix = cell - iy * CROP_W
            ys = iy.astype(jnp.float32) / jnp.float32(CROP_H - 1)
            xs = ix.astype(jnp.float32) / jnp.float32(CROP_W - 1)
            in_y = (y1 + ys * (y2 - y1)) * fH
            in_x = (x1 + xs * (x2 - x1)) * fW
            valid = ((in_y >= 0.0) & (in_y <= fH)
                     & (in_x >= 0.0) & (in_x <= fW))

            ty0 = in_y.astype(jnp.int32)
            tf0 = ty0.astype(jnp.float32)
            neg = in_y < tf0
            fy_i = jnp.where(neg, ty0 - 1, ty0)
            fy_f = jnp.where(neg, tf0 - 1.0, tf0)
            ly = in_y - fy_f
            ti = jnp.clip(fy_i, 0, H - 1)
            bo = jnp.clip(fy_i + 1, 0, H - 1)

            tx0 = in_x.astype(jnp.int32)
            xf0 = tx0.astype(jnp.float32)
            negx = in_x < xf0
            fx_i = jnp.where(negx, tx0 - 1, tx0)
            fx_f = jnp.where(negx, xf0 - 1.0, xf0)
            lx = in_x - fx_f
            li = jnp.clip(fx_i, 0, W - 1)
            ri = jnp.clip(fx_i + 1, 0, W - 1)

            vf = jnp.where(valid, 1.0, 0.0).astype(jnp.float32)
            omy = 1.0 - ly
            omx = 1.0 - lx
            trow = bi * (H * W) + ti * W
            brow = bi * (H * W) + bo * W
            rows = (trow + li, trow + ri, brow + li, brow + ri)
            ws = (omy * omx * vf, omy * lx * vf, ly * omx * vf, ly * lx * vf)
            return rows, ws

        def stage_group(g, box, idxr):
            cell = jnp.broadcast_to(g * L, (L,)) + lane
            rows, ws = cell_geom(cell, box)
            for cidx in range(4):
                wb_v[cidx, pl.ds(g * L, L)] = ws[cidx]
                idxr[pl.ds((g % 2) * 4 * L + cidx * L, L)] = rows[cidx]

        def stage_a(b):
            box = load_box(b)
            stage_group(0, box, idxa)
            stage_group(1, box, idxa)
            pltpu.async_copy(table_hbm.at[idxa], rowsa, sema)

        def stage_b(box):
            stage_group(2, box, idxb)
            c48 = jnp.broadcast_to(48, (L,))
            rows, ws48 = cell_geom(c48, box)
            c01 = jnp.where(lane == 0, rows[0], rows[1])
            c23 = jnp.where(lane == 2, rows[2], rows[3])
            idxb[pl.ds(4 * L, L)] = jnp.where(lane < 2, c01, c23)
            pltpu.async_copy(table_hbm.at[idxb], rowsb, semb)
            return ws48

        def compute_group(g, rowsr):
            o = (g % 2) * 4 * L

            def cell_body(j, c, g=g, o=o):
                col = jnp.broadcast_to(g * L, (L,)) + j
                wtl = plsc.load_gather(wb_v, [zero, col])
                wtr = plsc.load_gather(wb_v, [zero + 1, col])
                wbl = plsc.load_gather(wb_v, [zero + 2, col])
                wbr = plsc.load_gather(wb_v, [zero + 3, col])
                off = lane49 + col
                for kk in range(nvr):
                    sl = pl.ds(kk * L, L)
                    val = (wtl * rowsr[o + j, sl]
                           + wtr * rowsr[o + j + L, sl]
                           + wbl * rowsr[o + j + 2 * L, sl]
                           + wbr * rowsr[o + j + 3 * L, sl])
                    plsc.store_scatter(out_v, [off >> 7, off & 127], val)
                    off = off + L * NCELL
                return c

            lax.fori_loop(0, L, cell_body, 0)

        stage_a(0)

        def box_body(i, carry):
            b = jnp.maximum(i - 1, 0)
            box = load_box(b)
            pltpu.make_async_copy(table_hbm.at[idxa], rowsa, sema).wait()
            ws48 = stage_b(box)
            compute_group(0, rowsa)
            compute_group(1, rowsa)
            pltpu.make_async_copy(table_hbm.at[idxb], rowsb, semb).wait()
            stage_a(jnp.minimum(i, npb - 1))
            compute_group(2, rowsb)
            off48 = lane49 + 48
            for kk in range(nvr):
                sl = pl.ds(kk * L, L)
                val = (ws48[0] * rowsb[4 * L, sl]
                       + ws48[1] * rowsb[4 * L + 1, sl]
                       + ws48[2] * rowsb[4 * L + 2, sl]
                       + ws48[3] * rowsb[4 * L + 3, sl])
                plsc.store_scatter(out_v, [off48 >> 7, off48 & 127], val)
                off48 = off48 + L * NCELL

            n = wid * npb + b

            @pl.when((n < N) & (i > 0))
            def _():
                pltpu.sync_copy(out_v, out_hbm.at[n])

            return carry

        lax.fori_loop(0, npb + 1, box_body, 0)
        pltpu.make_async_copy(table_hbm.at[idxa], rowsa, sema).wait()

    return k(table, boxes_pad)


def kernel(image, boxes, box_ind):
    B, C, H, W = image.shape
    N = boxes.shape[0]
    nw = 32
    npb = -(-N // (nw * 8)) * 8
    npad = npb * nw

    table = jnp.transpose(image, (0, 2, 3, 1)).reshape(B * H * W, C)
    boxes5 = jnp.concatenate(
        [boxes, box_ind[:, None].astype(jnp.float32)], axis=1)
    boxes_pad = jnp.zeros((npad, L), jnp.float32).at[:N, :5].set(boxes5)

    out = _sc_crop_and_resize(table, boxes_pad, B, C, H, W, N, npb, nw)
    return out.reshape(N, C, CROP_H, CROP_W)
